# baseline (speedup 1.0000x reference)
import jax
import jax.numpy as jnp
from jax import lax
from jax.experimental import pallas as pl
from jax.experimental.pallas import tpu as pltpu
from jax.experimental.pallas import tpu_sc as plsc

D = 32
H = 768
NC = 2
NS = 16
NW = NC * NS
CH = 128
SPG = 2
GROUP = CH * SPG
PACK = CH // D


def _gather_extract(ids_ref, tab_ref, out_ref, n, wid,
                    ids_v, sidx, cbase, gbuf, obuf, sem):
  per_w = n // NW
  groups = per_w // GROUP

  @pl.loop(0, groups)
  def _g(g):
    idrow = wid * (per_w // CH) + g * SPG
    pltpu.sync_copy(ids_ref.at[pl.ds(idrow, SPG)], ids_v)
    for j in range(SPG):
      row = ids_v.at[j]
      for t in range(CH // 16):
        raw = row[pl.ds(t * 16, 16)]
        sidx[pl.ds(j * CH + t * 16, 16)] = lax.shift_right_logical(raw, 2)
        cbase[pl.ds(j * CH + t * 16, 16)] = (raw & 3) * D
    cps = [pltpu.async_copy(tab_ref.at[sidx.at[pl.ds(j * CH, CH)]],
                            gbuf.at[pl.ds(j * CH, CH)], sem)
           for j in range(SPG)]
    for cp in cps:
      cp.wait()
    @pl.loop(0, GROUP // 16)
    def _t(t):
      r0 = pl.multiple_of(t * 16, 16)
      rows = r0 + jnp.arange(16, dtype=jnp.int32)
      cb = cbase[pl.ds(r0, 16)]
      pbase = rows * D
      for c in range(D):
        v = plsc.load_gather(gbuf, [rows, cb + c])
        p = pbase + c
        plsc.store_scatter(obuf, [lax.shift_right_logical(p, 7), p & 127], v)
    orow = wid * (per_w * D // CH) + g * (GROUP * D // CH)
    pltpu.sync_copy(obuf, out_ref.at[pl.ds(orow, GROUP * D // CH)])


def _sc_gather(item_idx, user_idx, item_tab, user_tab, ni, nu):
  mesh = plsc.VectorSubcoreMesh(core_axis_name="c", subcore_axis_name="s")

  def body(item_idx_ref, user_idx_ref, item_tab_ref, user_tab_ref,
           items_out, users_out, ids_v, sidx, cbase, gbuf, obuf, sem):
    wid = lax.axis_index("s") * NC + lax.axis_index("c")
    _gather_extract(item_idx_ref, item_tab_ref, items_out, ni, wid,
                    ids_v, sidx, cbase, gbuf, obuf, sem)
    _gather_extract(user_idx_ref, user_tab_ref, users_out, nu, wid,
                    ids_v, sidx, cbase, gbuf, obuf, sem)

  fn = pl.kernel(
      body,
      out_type=(jax.ShapeDtypeStruct((ni * D // CH, CH), jnp.float32),
                jax.ShapeDtypeStruct((nu * D // CH, CH), jnp.float32)),
      mesh=mesh,
      compiler_params=pltpu.CompilerParams(use_tc_tiling_on_sc=True,
                                           needs_layout_passes=False),
      scratch_types=[
          pltpu.VMEM((SPG, CH), jnp.int32),
          pltpu.VMEM((GROUP,), jnp.int32),
          pltpu.VMEM((GROUP,), jnp.int32),
          pltpu.VMEM((GROUP, CH), jnp.float32),
          pltpu.VMEM((GROUP * D // CH, CH), jnp.float32),
          pltpu.SemaphoreType.DMA,
      ],
  )
  return fn(item_idx, user_idx, item_tab, user_tab)


def _project_packed(x4, wblk, bm):
  m4 = x4.shape[0]

  def mm(x_ref, w_ref, o_ref):
    o_ref[...] = lax.dot_general(x_ref[...], w_ref[...],
                                 (((1,), (1,)), ((), ())),
                                 preferred_element_type=jnp.float32)

  return pl.pallas_call(
      mm,
      grid=(m4 // bm,),
      in_specs=[pl.BlockSpec((bm, 4 * D), lambda i: (i, 0)),
                pl.BlockSpec((4 * H, 4 * D), lambda i: (0, 0))],
      out_specs=pl.BlockSpec((bm, 4 * H), lambda i: (i, 0)),
      out_shape=jax.ShapeDtypeStruct((m4, 4 * H), jnp.float32),
  )(x4, wblk)


def _block_diag4(w):
  z = jnp.zeros((H, D), w.dtype)
  rows = [jnp.concatenate([w if i == j else z for j in range(4)], axis=1)
          for i in range(4)]
  return jnp.concatenate(rows, axis=0)


def kernel(user_ids, item_ids, user_table, item_table, W_user, W_item):
  b, l = item_ids.shape
  ni = b * l
  nv = item_table.shape[0]
  item_idx = item_ids.reshape(ni // CH, CH)
  user_idx = user_ids.reshape(b // CH, CH)
  items_f, users_f = _sc_gather(item_idx, user_idx,
                                item_table.reshape(nv // PACK, CH),
                                user_table.reshape(nv // PACK, CH),
                                ni, b)
  wblk_u = _block_diag4(W_user)
  wblk_i = _block_diag4(W_item)
  u_proj = _project_packed(users_f, wblk_u, 512).reshape(b, H)
  i_proj = _project_packed(items_f, wblk_i, 1024).reshape(b, l, H)
  return (u_proj, i_proj)
